# baseline (speedup 1.0000x reference)
import jax
import jax.numpy as jnp
from jax import lax
from jax.experimental import pallas as pl
from jax.experimental.pallas import tpu as pltpu
from jax.experimental.pallas import tpu_sc as plsc

B = 1024
S = 200
D = 128
N = B * S

NUM_CORES = 2
NUM_SUBCORES = 16
LANES = 16
NW = NUM_CORES * NUM_SUBCORES

TOK_PER_W = N // NW
T = 128
CHUNKS = TOK_PER_W // T
NBUF = 3
FULL_ROUNDS = CHUNKS // NBUF
TAIL = CHUNKS - FULL_ROUNDS * NBUF


def _sc_body(in_hbm, cat_hbm, mask_hbm, table_hbm, out_hbm,
             idx_all, mask_all, in_v, emb_v,
             in_sem, gat_sem, out_sem):
    wid = lax.axis_index("s") * NUM_CORES + lax.axis_index("c")
    tok0 = wid * TOK_PER_W

    pltpu.sync_copy(cat_hbm.at[wid], idx_all)
    pltpu.sync_copy(mask_hbm.at[wid], mask_all)

    def start_fetch(ci, b):
        base = tok0 + ci * T
        pltpu.async_copy(table_hbm.at[idx_all.at[ci]], emb_v[b], gat_sem[b])
        pltpu.async_copy(in_hbm.at[pl.ds(base, T)], in_v[b], in_sem[b])

    def wait_fetch(ci, b):
        base = tok0 + ci * T
        pltpu.make_async_copy(
            table_hbm.at[idx_all.at[ci]], emb_v[b], gat_sem[b]).wait()
        pltpu.make_async_copy(
            in_hbm.at[pl.ds(base, T)], in_v[b], in_sem[b]).wait()

    def wait_out(ci, b):
        base = tok0 + ci * T
        pltpu.make_async_copy(
            in_v[b], out_hbm.at[pl.ds(base, T)], out_sem[b]).wait()

    def compute_chunk(ci, b):
        def grp_body(g, c):
            m16 = mask_all[ci, pl.ds(g * LANES, LANES)]
            for j in range(LANES):
                keep = m16[j] == 0
                t = g * LANES + j
                for d in range(D // LANES):
                    x = in_v[b][t, pl.ds(d * LANES, LANES)]
                    e = emb_v[b][t, pl.ds(d * LANES, LANES)]
                    in_v[b][t, pl.ds(d * LANES, LANES)] = x * jnp.where(
                        keep, e, jnp.float32(1.0))
            return c

        lax.fori_loop(0, T // LANES, grp_body, 0)

    for ci in range(NBUF - 1):
        start_fetch(ci, ci)

    def ring_body(cp, carry):
        for b in range(NBUF):
            ci = cp * NBUF + b
            nb = (b + NBUF - 1) % NBUF

            @pl.when(ci >= 1)
            def _():
                wait_out(ci - 1, nb)

            @pl.when(ci + NBUF - 1 < CHUNKS)
            def _():
                start_fetch(ci + NBUF - 1, nb)

            wait_fetch(ci, b)
            compute_chunk(ci, b)
            base = tok0 + ci * T
            pltpu.async_copy(in_v[b], out_hbm.at[pl.ds(base, T)], out_sem[b])
        return carry

    lax.fori_loop(0, FULL_ROUNDS, ring_body, 0)
    for ci in range(FULL_ROUNDS * NBUF, CHUNKS):
        b = ci % NBUF
        nb = (b + NBUF - 1) % NBUF
        wait_out(ci - 1, nb)
        wait_fetch(ci, b)
        compute_chunk(ci, b)
        base = tok0 + ci * T
        pltpu.async_copy(in_v[b], out_hbm.at[pl.ds(base, T)], out_sem[b])
    wait_out(CHUNKS - 1, (CHUNKS - 1) % NBUF)


@jax.jit
def _run(in_flat, cats, mask, table):
    mesh = plsc.VectorSubcoreMesh(
        core_axis_name="c", subcore_axis_name="s",
        num_cores=NUM_CORES, num_subcores=NUM_SUBCORES)
    fn = pl.kernel(
        _sc_body,
        out_type=jax.ShapeDtypeStruct((N, D), jnp.float32),
        mesh=mesh,
        scratch_types=[
            pltpu.VMEM((CHUNKS, T), jnp.int32),
            pltpu.VMEM((CHUNKS, T), jnp.int32),
            [pltpu.VMEM((T, D), jnp.float32) for _ in range(NBUF)],
            [pltpu.VMEM((T, D), jnp.float32) for _ in range(NBUF)],
            [pltpu.SemaphoreType.DMA for _ in range(NBUF)],
            [pltpu.SemaphoreType.DMA for _ in range(NBUF)],
            [pltpu.SemaphoreType.DMA for _ in range(NBUF)],
        ],
    )
    return fn(in_flat, cats, mask, table)


def kernel(inputs, categories, mask_positions, category_embedding):
    in_flat = inputs.reshape(N, D)
    cats = categories.reshape(NW, CHUNKS, T).astype(jnp.int32)
    mask = mask_positions.reshape(NW, CHUNKS, T).astype(jnp.int32)
    out = _run(in_flat, cats, mask, category_embedding)
    return out.reshape(B, S, D)


# speedup vs baseline: 1.4975x; 1.4975x over previous
import jax
import jax.numpy as jnp
from jax import lax
from jax.experimental import pallas as pl
from jax.experimental.pallas import tpu as pltpu
from jax.experimental.pallas import tpu_sc as plsc

B = 1024
S = 200
D = 128
N = B * S

NUM_CORES = 2
NUM_SUBCORES = 16
LANES = 16
NW = NUM_CORES * NUM_SUBCORES

TOK_PER_W = N // NW
T = 128
CHUNKS = TOK_PER_W // T
NBUF = 2


def _sc_body(in_hbm, cat_hbm, mask_hbm, table_hbm, out_hbm,
             idx_all, mask_all, in_v, emb_v, out_v,
             in_sem, gat_sem, out_sem):
    wid = lax.axis_index("s") * NUM_CORES + lax.axis_index("c")
    tok0 = wid * TOK_PER_W

    pltpu.sync_copy(cat_hbm.at[wid], idx_all)
    pltpu.sync_copy(mask_hbm.at[wid], mask_all)

    splat_idx = [jnp.full((LANES, 1), j, jnp.int32) for j in range(LANES)]
    ones_v = jnp.ones((LANES,), jnp.float32)
    splat_dnums = lax.GatherDimensionNumbers(
        offset_dims=(), collapsed_slice_dims=(0,), start_index_map=(0,))

    def splat(v16, j):
        return lax.gather(v16, splat_idx[j], splat_dnums, (1,),
                          mode=lax.GatherScatterMode.PROMISE_IN_BOUNDS)

    def start_fetch(ci, b):
        base = tok0 + ci * T
        pltpu.async_copy(in_hbm.at[pl.ds(base, T)], in_v[b], in_sem[b])
        pltpu.async_copy(table_hbm.at[idx_all.at[ci]], emb_v[b], gat_sem[b])

    def wait_fetch(ci, b):
        base = tok0 + ci * T
        pltpu.make_async_copy(
            in_hbm.at[pl.ds(base, T)], in_v[b], in_sem[b]).wait()
        pltpu.make_async_copy(
            table_hbm.at[idx_all.at[ci]], emb_v[b], gat_sem[b]).wait()

    def wait_out(ci, b):
        base = tok0 + ci * T
        pltpu.make_async_copy(
            out_v[b], out_hbm.at[pl.ds(base, T)], out_sem[b]).wait()

    start_fetch(0, 0)

    def pair_body(cp, carry):
        for b in range(NBUF):
            ci = cp * NBUF + b
            nb = (b + 1) % NBUF

            @pl.when(ci + 1 < CHUNKS)
            def _():
                start_fetch(ci + 1, nb)

            wait_fetch(ci, b)

            def grp_body(g, c):
                fm16 = mask_all[ci, pl.ds(g * LANES, LANES)].astype(
                    jnp.float32)
                om16 = ones_v - fm16
                for j in range(LANES):
                    fm = splat(fm16, j)
                    om = splat(om16, j)
                    t = g * LANES + j
                    for d in range(D // LANES):
                        x = in_v[b][t, pl.ds(d * LANES, LANES)]
                        e = emb_v[b][t, pl.ds(d * LANES, LANES)]
                        out_v[b][t, pl.ds(d * LANES, LANES)] = x * (
                            e * om + fm)
                return c

            lax.fori_loop(0, T // LANES, grp_body, 0)

            @pl.when(ci >= 1)
            def _():
                wait_out(ci - 1, nb)

            base = tok0 + ci * T
            pltpu.async_copy(out_v[b], out_hbm.at[pl.ds(base, T)], out_sem[b])
        return carry

    lax.fori_loop(0, CHUNKS // NBUF, pair_body, 0)
    wait_out(CHUNKS - 1, (CHUNKS - 1) % NBUF)


@jax.jit
def _run(in_flat, cats, mask, table):
    mesh = plsc.VectorSubcoreMesh(
        core_axis_name="c", subcore_axis_name="s",
        num_cores=NUM_CORES, num_subcores=NUM_SUBCORES)
    fn = pl.kernel(
        _sc_body,
        out_type=jax.ShapeDtypeStruct((N, D), jnp.float32),
        mesh=mesh,
        scratch_types=[
            pltpu.VMEM((CHUNKS, T), jnp.int32),
            pltpu.VMEM((CHUNKS, T), jnp.int32),
            [pltpu.VMEM((T, D), jnp.float32) for _ in range(NBUF)],
            [pltpu.VMEM((T, D), jnp.float32) for _ in range(NBUF)],
            [pltpu.VMEM((T, D), jnp.float32) for _ in range(NBUF)],
            [pltpu.SemaphoreType.DMA for _ in range(NBUF)],
            [pltpu.SemaphoreType.DMA for _ in range(NBUF)],
            [pltpu.SemaphoreType.DMA for _ in range(NBUF)],
        ],
    )
    return fn(in_flat, cats, mask, table)


def kernel(inputs, categories, mask_positions, category_embedding):
    in_flat = inputs.reshape(N, D)
    cats = categories.reshape(NW, CHUNKS, T).astype(jnp.int32)
    mask = (mask_positions.reshape(NW, CHUNKS, T) != 0).astype(jnp.int32)
    out = _run(in_flat, cats, mask, category_embedding)
    return out.reshape(B, S, D)


# speedup vs baseline: 1.6428x; 1.0970x over previous
import jax
import jax.numpy as jnp
from jax import lax
from jax.experimental import pallas as pl
from jax.experimental.pallas import tpu as pltpu
from jax.experimental.pallas import tpu_sc as plsc

B = 1024
S = 200
D = 128
N = B * S

NUM_CORES = 2
NUM_SUBCORES = 16
LANES = 16
NW = NUM_CORES * NUM_SUBCORES

TOK_PER_W = N // NW
T = 128
CHUNKS = TOK_PER_W // T
NBUF = 3
FULL_ROUNDS = CHUNKS // NBUF


def _sc_body(in_hbm, cat_hbm, mask_hbm, table_hbm, out_hbm,
             idx_all, mask_all, in_v, emb_v,
             in_sem, gat_sem, out_sem):
    wid = lax.axis_index("s") * NUM_CORES + lax.axis_index("c")
    tok0 = wid * TOK_PER_W

    pltpu.sync_copy(cat_hbm.at[wid], idx_all)
    pltpu.sync_copy(mask_hbm.at[wid], mask_all)

    splat_idx = [jnp.full((LANES, 1), j, jnp.int32) for j in range(LANES)]
    ones_v = jnp.ones((LANES,), jnp.float32)
    splat_dnums = lax.GatherDimensionNumbers(
        offset_dims=(), collapsed_slice_dims=(0,), start_index_map=(0,))

    def splat(v16, j):
        return lax.gather(v16, splat_idx[j], splat_dnums, (1,),
                          mode=lax.GatherScatterMode.PROMISE_IN_BOUNDS)

    def start_fetch(ci, b):
        base = tok0 + ci * T
        pltpu.async_copy(in_hbm.at[pl.ds(base, T)], in_v[b], in_sem[b])
        pltpu.async_copy(table_hbm.at[idx_all.at[ci]], emb_v[b], gat_sem[b])

    def wait_fetch(ci, b):
        base = tok0 + ci * T
        pltpu.make_async_copy(
            in_hbm.at[pl.ds(base, T)], in_v[b], in_sem[b]).wait()
        pltpu.make_async_copy(
            table_hbm.at[idx_all.at[ci]], emb_v[b], gat_sem[b]).wait()

    def wait_out(ci, b):
        base = tok0 + ci * T
        pltpu.make_async_copy(
            in_v[b], out_hbm.at[pl.ds(base, T)], out_sem[b]).wait()

    def compute_chunk(ci, b):
        def grp_body(g, c):
            fm16 = mask_all[ci, pl.ds(g * LANES, LANES)].astype(jnp.float32)
            om16 = ones_v - fm16
            for j in range(LANES):
                fm = splat(fm16, j)
                om = splat(om16, j)
                t = g * LANES + j
                for d in range(D // LANES):
                    x = in_v[b][t, pl.ds(d * LANES, LANES)]
                    e = emb_v[b][t, pl.ds(d * LANES, LANES)]
                    in_v[b][t, pl.ds(d * LANES, LANES)] = x * (e * om + fm)
            return c

        lax.fori_loop(0, T // LANES, grp_body, 0)

    def start_out(ci, b):
        base = tok0 + ci * T
        pltpu.async_copy(in_v[b], out_hbm.at[pl.ds(base, T)], out_sem[b])

    for ci in range(NBUF - 1):
        start_fetch(ci, ci)

    def ring_body(cp, carry):
        for b in range(NBUF):
            ci = cp * NBUF + b
            nb = (b + NBUF - 1) % NBUF

            @pl.when(ci >= 1)
            def _():
                wait_out(ci - 1, nb)

            @pl.when(ci + NBUF - 1 < CHUNKS)
            def _():
                start_fetch(ci + NBUF - 1, nb)

            wait_fetch(ci, b)
            compute_chunk(ci, b)
            start_out(ci, b)
        return carry

    lax.fori_loop(0, FULL_ROUNDS, ring_body, 0)
    for ci in range(FULL_ROUNDS * NBUF, CHUNKS):
        b = ci % NBUF
        wait_out(ci - 1, (b + NBUF - 1) % NBUF)
        wait_fetch(ci, b)
        compute_chunk(ci, b)
        start_out(ci, b)
    wait_out(CHUNKS - 1, (CHUNKS - 1) % NBUF)


@jax.jit
def _run(in_flat, cats, mask, table):
    mesh = plsc.VectorSubcoreMesh(
        core_axis_name="c", subcore_axis_name="s",
        num_cores=NUM_CORES, num_subcores=NUM_SUBCORES)
    fn = pl.kernel(
        _sc_body,
        out_type=jax.ShapeDtypeStruct((N, D), jnp.float32),
        mesh=mesh,
        scratch_types=[
            pltpu.VMEM((CHUNKS, T), jnp.int32),
            pltpu.VMEM((CHUNKS, T), jnp.int32),
            [pltpu.VMEM((T, D), jnp.float32) for _ in range(NBUF)],
            [pltpu.VMEM((T, D), jnp.float32) for _ in range(NBUF)],
            [pltpu.SemaphoreType.DMA for _ in range(NBUF)],
            [pltpu.SemaphoreType.DMA for _ in range(NBUF)],
            [pltpu.SemaphoreType.DMA for _ in range(NBUF)],
        ],
    )
    return fn(in_flat, cats, mask, table)


def kernel(inputs, categories, mask_positions, category_embedding):
    in_flat = inputs.reshape(N, D)
    cats = categories.reshape(NW, CHUNKS, T).astype(jnp.int32)
    mask = (mask_positions.reshape(NW, CHUNKS, T) != 0).astype(jnp.int32)
    out = _run(in_flat, cats, mask, category_embedding)
    return out.reshape(B, S, D)


# speedup vs baseline: 1.7219x; 1.0481x over previous
import jax
import jax.numpy as jnp
from jax import lax
from jax.experimental import pallas as pl
from jax.experimental.pallas import tpu as pltpu
from jax.experimental.pallas import tpu_sc as plsc

B = 1024
S = 200
D = 128
N = B * S

NUM_CORES = 2
NUM_SUBCORES = 16
LANES = 16
NW = NUM_CORES * NUM_SUBCORES

TOK_PER_W = N // NW
T = 160
CHUNKS = TOK_PER_W // T
NBUF = 2
FULL_ROUNDS = CHUNKS // NBUF


def _sc_body(in_hbm, cat_hbm, mask_hbm, table_hbm, out_hbm,
             idx_all, mask_all, in_v, emb_v,
             in_sem, gat_sem, gat2_sem, out_sem):
    wid = lax.axis_index("s") * NUM_CORES + lax.axis_index("c")
    tok0 = wid * TOK_PER_W

    pltpu.sync_copy(cat_hbm.at[wid], idx_all)
    pltpu.sync_copy(mask_hbm.at[wid], mask_all)

    splat_idx = [jnp.full((LANES, 1), j, jnp.int32) for j in range(LANES)]
    ones_v = jnp.ones((LANES,), jnp.float32)
    splat_dnums = lax.GatherDimensionNumbers(
        offset_dims=(), collapsed_slice_dims=(0,), start_index_map=(0,))

    def splat(v16, j):
        return lax.gather(v16, splat_idx[j], splat_dnums, (1,),
                          mode=lax.GatherScatterMode.PROMISE_IN_BOUNDS)

    def start_fetch(ci, b):
        base = tok0 + ci * T
        pltpu.async_copy(in_hbm.at[pl.ds(base, T)], in_v[b], in_sem[b])
        pltpu.async_copy(table_hbm.at[idx_all.at[ci, pl.ds(0, 128)]],
                         emb_v[b].at[pl.ds(0, 128)], gat_sem[b])
        pltpu.async_copy(table_hbm.at[idx_all.at[ci, pl.ds(128, T - 128)]],
                         emb_v[b].at[pl.ds(128, T - 128)], gat2_sem[b])

    def wait_fetch(ci, b):
        base = tok0 + ci * T
        pltpu.make_async_copy(
            in_hbm.at[pl.ds(base, T)], in_v[b], in_sem[b]).wait()
        pltpu.make_async_copy(
            table_hbm.at[idx_all.at[ci, pl.ds(0, 128)]],
            emb_v[b].at[pl.ds(0, 128)], gat_sem[b]).wait()
        pltpu.make_async_copy(
            table_hbm.at[idx_all.at[ci, pl.ds(128, T - 128)]],
            emb_v[b].at[pl.ds(128, T - 128)], gat2_sem[b]).wait()

    def wait_out(ci, b):
        base = tok0 + ci * T
        pltpu.make_async_copy(
            in_v[b], out_hbm.at[pl.ds(base, T)], out_sem[b]).wait()

    def compute_chunk(ci, b):
        def grp_body(g, c):
            fm16 = jnp.minimum(
                mask_all[ci, pl.ds(g * LANES, LANES)], 1).astype(jnp.float32)
            om16 = ones_v - fm16
            for j in range(LANES):
                fm = splat(fm16, j)
                om = splat(om16, j)
                t = g * LANES + j
                for d in range(D // LANES):
                    x = in_v[b][t, pl.ds(d * LANES, LANES)]
                    e = emb_v[b][t, pl.ds(d * LANES, LANES)]
                    in_v[b][t, pl.ds(d * LANES, LANES)] = x * (e * om + fm)
            return c

        lax.fori_loop(0, T // LANES, grp_body, 0)

    def start_out(ci, b):
        base = tok0 + ci * T
        pltpu.async_copy(in_v[b], out_hbm.at[pl.ds(base, T)], out_sem[b])

    for ci in range(NBUF - 1):
        start_fetch(ci, ci)

    def ring_body(cp, carry):
        for b in range(NBUF):
            ci = cp * NBUF + b
            nb = (b + NBUF - 1) % NBUF

            @pl.when(ci >= 1)
            def _():
                wait_out(ci - 1, nb)

            @pl.when(ci + NBUF - 1 < CHUNKS)
            def _():
                start_fetch(ci + NBUF - 1, nb)

            wait_fetch(ci, b)
            compute_chunk(ci, b)
            start_out(ci, b)
        return carry

    lax.fori_loop(0, FULL_ROUNDS, ring_body, 0)
    for ci in range(FULL_ROUNDS * NBUF, CHUNKS):
        b = ci % NBUF
        wait_out(ci - 1, (b + NBUF - 1) % NBUF)
        wait_fetch(ci, b)
        compute_chunk(ci, b)
        start_out(ci, b)
    wait_out(CHUNKS - 1, (CHUNKS - 1) % NBUF)


@jax.jit
def _run(in_flat, cats, mask, table):
    mesh = plsc.VectorSubcoreMesh(
        core_axis_name="c", subcore_axis_name="s",
        num_cores=NUM_CORES, num_subcores=NUM_SUBCORES)
    fn = pl.kernel(
        _sc_body,
        out_type=jax.ShapeDtypeStruct((N, D), jnp.float32),
        mesh=mesh,
        scratch_types=[
            pltpu.VMEM((CHUNKS, T), jnp.int32),
            pltpu.VMEM((CHUNKS, T), jnp.int32),
            [pltpu.VMEM((T, D), jnp.float32) for _ in range(NBUF)],
            [pltpu.VMEM((T, D), jnp.float32) for _ in range(NBUF)],
            [pltpu.SemaphoreType.DMA for _ in range(NBUF)],
            [pltpu.SemaphoreType.DMA for _ in range(NBUF)],
            [pltpu.SemaphoreType.DMA for _ in range(NBUF)],
            [pltpu.SemaphoreType.DMA for _ in range(NBUF)],
        ],
    )
    return fn(in_flat, cats, mask, table)


def kernel(inputs, categories, mask_positions, category_embedding):
    in_flat = inputs.reshape(N, D)
    cats = categories.reshape(NW, CHUNKS, T).astype(jnp.int32)
    mask = mask_positions.reshape(NW, CHUNKS, T).astype(jnp.int32)
    out = _run(in_flat, cats, mask, category_embedding)
    return out.reshape(B, S, D)
